# baseline (speedup 1.0000x reference)
import functools

import jax
import jax.numpy as jnp
from jax import lax
from jax.experimental import pallas as pl
from jax.experimental.pallas import tpu as pltpu
from jax.experimental.pallas import tpu_sc as plsc

B = 16384
DG = 16
DM = 64

NC = 2
NS = 16
NW = NC * NS
BPW = B // NW
CHUNK = 128
NCHUNK = BPW // CHUNK


def _sc_gather_body(uid_hbm, iid_hbm, eug_hbm, eig_hbm, eum_hbm, eim_hbm,
                    ug_out, ig_out, um_out, im_out,
                    uidx_v, iidx_v, ug_v, ig_v, um_v, im_v, sem):
  wid = lax.axis_index("s") * NC + lax.axis_index("c")
  base = wid * BPW
  pltpu.sync_copy(uid_hbm.at[pl.ds(wid * NCHUNK, NCHUNK), :], uidx_v)
  pltpu.sync_copy(iid_hbm.at[pl.ds(wid * NCHUNK, NCHUNK), :], iidx_v)
  copies = []
  for j in range(NCHUNK):
    sl = pl.ds(j * CHUNK, CHUNK)
    copies.append(pltpu.async_copy(eug_hbm.at[uidx_v.at[j]], ug_v.at[sl, :], sem))
    copies.append(pltpu.async_copy(eig_hbm.at[iidx_v.at[j]], ig_v.at[sl, :], sem))
    copies.append(pltpu.async_copy(eum_hbm.at[uidx_v.at[j]], um_v.at[sl, :], sem))
    copies.append(pltpu.async_copy(eim_hbm.at[iidx_v.at[j]], im_v.at[sl, :], sem))
  for c in copies:
    c.wait()
  pltpu.sync_copy(ug_v, ug_out.at[pl.ds(base, BPW), :])
  pltpu.sync_copy(ig_v, ig_out.at[pl.ds(base, BPW), :])
  pltpu.sync_copy(um_v, um_out.at[pl.ds(base, BPW), :])
  pltpu.sync_copy(im_v, im_out.at[pl.ds(base, BPW), :])


_sc_gather = functools.partial(
    pl.kernel,
    out_type=[
        jax.ShapeDtypeStruct((B, DG), jnp.float32),
        jax.ShapeDtypeStruct((B, DG), jnp.float32),
        jax.ShapeDtypeStruct((B, DM), jnp.float32),
        jax.ShapeDtypeStruct((B, DM), jnp.float32),
    ],
    mesh=plsc.VectorSubcoreMesh(core_axis_name="c", subcore_axis_name="s"),
    scratch_types=[
        pltpu.VMEM((NCHUNK, CHUNK), jnp.int32),
        pltpu.VMEM((NCHUNK, CHUNK), jnp.int32),
        pltpu.VMEM((BPW, DG), jnp.float32),
        pltpu.VMEM((BPW, DG), jnp.float32),
        pltpu.VMEM((BPW, DM), jnp.float32),
        pltpu.VMEM((BPW, DM), jnp.float32),
        pltpu.SemaphoreType.DMA,
    ],
    compiler_params=pltpu.CompilerParams(use_tc_tiling_on_sc=False),
)(_sc_gather_body)


TC_BLK = 1024


def _tc_mlp_body(ug_ref, ig_ref, um_ref, im_ref,
                 w0_ref, b0_ref, w1_ref, b1_ref, w2_ref, b2_ref,
                 wf_ref, bf_ref, out_ref):
  w0 = w0_ref[...]
  h = jnp.dot(um_ref[...], w0[:DM], preferred_element_type=jnp.float32)
  h += jnp.dot(im_ref[...], w0[DM:], preferred_element_type=jnp.float32)
  h = jnp.maximum(h + b0_ref[...], 0.0)
  h = jnp.maximum(
      jnp.dot(h, w1_ref[...], preferred_element_type=jnp.float32) + b1_ref[...], 0.0)
  h = jnp.maximum(
      jnp.dot(h, w2_ref[...], preferred_element_type=jnp.float32) + b2_ref[...], 0.0)
  g = ug_ref[...] * ig_ref[...]
  wf = wf_ref[...]
  res = jnp.dot(g, wf[:DG], preferred_element_type=jnp.float32)
  res += jnp.dot(h, wf[DG:], preferred_element_type=jnp.float32)
  out_ref[...] = res + bf_ref[...]


def _tc_mlp(ug, ig, um, im, W0, b0, W1, b1, W2, b2, Wf, bf):
  nblk = B // TC_BLK
  row_spec = lambda d: pl.BlockSpec((TC_BLK, d), lambda i: (i, 0))
  full_spec = lambda s: pl.BlockSpec(s, lambda i: tuple(0 for _ in s))
  return pl.pallas_call(
      _tc_mlp_body,
      grid=(nblk,),
      in_specs=[
          row_spec(DG), row_spec(DG), row_spec(DM), row_spec(DM),
          full_spec((2 * DM, DM)), full_spec((1, DM)),
          full_spec((DM, DM // 2)), full_spec((1, DM // 2)),
          full_spec((DM // 2, DM // 4)), full_spec((1, DM // 4)),
          full_spec((2 * DG, 1)), full_spec((1, 1)),
      ],
      out_specs=pl.BlockSpec((TC_BLK, 1), lambda i: (i, 0)),
      out_shape=jax.ShapeDtypeStruct((B, 1), jnp.float32),
  )(ug, ig, um, im, W0, b0.reshape(1, DM), W1, b1.reshape(1, DM // 2),
    W2, b2.reshape(1, DM // 4), Wf, bf.reshape(1, 1))


@jax.jit
def kernel(x, eu_gmf, ei_gmf, eu_mlp, ei_mlp, W0, b0, W1, b1, W2, b2, Wf, bf):
  uid = x[:, 0].reshape(B // CHUNK, CHUNK)
  iid = x[:, 1].reshape(B // CHUNK, CHUNK)
  ug, ig, um, im = _sc_gather(uid, iid, eu_gmf, ei_gmf, eu_mlp, ei_mlp)
  out = _tc_mlp(ug, ig, um, im, W0, b0, W1, b1, W2, b2, Wf, bf)
  return out.reshape(-1)
